# baseline (speedup 1.0000x reference)
import functools

import jax
import jax.numpy as jnp
import numpy as np
from jax import lax
from jax.experimental import pallas as pl
from jax.experimental.pallas import tpu as pltpu
from jax.experimental.pallas import tpu_sc as plsc

K_EMBED = 8192
D_EMBED = 256
COMMIT_W = 0.25

BATCH = 8
SEQ = 1024
N_TOK = BATCH * SEQ

BT = 1024
BK = 1024
TB = N_TOK // BT
KB = K_EMBED // BK

_REV = np.zeros(BK, dtype=np.int32)
for _r in range(BK):
    _b = 0
    for _k in range(10):
        _b = (_b << 1) | ((_r >> _k) & 1)
    _REV[_r] = _b
_PERM = (np.arange(K_EMBED, dtype=np.int32) // BK) * BK + np.tile(_REV, K_EMBED // BK)
_REVCOL = _REV.astype(np.float32).reshape(BK, 1)

_NC = 2
_NS = 16
_NW = _NC * _NS
_BPW = N_TOK // _NW
_IDX_CHUNK = 128
_NCHUNK = _BPW // _IDX_CHUNK


def _argmin_body(x2_ref, e2_ref, x_ref, e_ref, rev_ref, idx_ref, lsum_ref,
                 best_ref, bidx_ref):
    tb = pl.program_id(0)
    kb = pl.program_id(1)

    mm = lax.dot_general(
        e_ref[...], x_ref[...],
        dimension_numbers=(((1,), (1,)), ((), ())),
        preferred_element_type=jnp.float32)
    s = x2_ref[...] + e2_ref[...]
    dist = s - 2.0 * mm

    v = dist
    i = jnp.broadcast_to(rev_ref[...], (BK, BT))
    h = BK
    while h > 1:
        h //= 2
        lo_v, hi_v = v[:h], v[h:]
        take = hi_v < lo_v
        v = jnp.where(take, hi_v, lo_v)
        i = jnp.where(take, i[h:], i[:h])
    minv = v
    midx = i + (kb * BK).astype(jnp.float32)

    @pl.when(kb == 0)
    def _():
        best_ref[...] = minv
        bidx_ref[...] = midx

    @pl.when(kb > 0)
    def _():
        upd = minv < best_ref[...]
        best_ref[...] = jnp.where(upd, minv, best_ref[...])
        bidx_ref[...] = jnp.where(upd, midx, bidx_ref[...])

    @pl.when((tb == 0) & (kb == 0))
    def _():
        lsum_ref[...] = jnp.zeros((1, 1), jnp.float32)

    @pl.when(kb == KB - 1)
    def _():
        idx_ref[...] = bidx_ref[...].astype(jnp.int32).reshape(1, 1, BT)
        lsum_ref[...] += jnp.sum(best_ref[...]).reshape(1, 1)


def _argmin_call(x2, e2, x_flat, emb):
    return pl.pallas_call(
        _argmin_body,
        grid=(TB, KB),
        in_specs=[
            pl.BlockSpec((1, BT), lambda tb, kb: (0, tb)),
            pl.BlockSpec((BK, 1), lambda tb, kb: (kb, 0)),
            pl.BlockSpec((BT, D_EMBED), lambda tb, kb: (tb, 0)),
            pl.BlockSpec((BK, D_EMBED), lambda tb, kb: (kb, 0)),
            pl.BlockSpec((BK, 1), lambda tb, kb: (0, 0)),
        ],
        out_specs=[
            pl.BlockSpec((1, 1, BT), lambda tb, kb: (tb, 0, 0)),
            pl.BlockSpec((1, 1), lambda tb, kb: (0, 0)),
        ],
        out_shape=[
            jax.ShapeDtypeStruct((TB, 1, BT), jnp.int32),
            jax.ShapeDtypeStruct((1, 1), jnp.float32),
        ],
        scratch_shapes=[
            pltpu.VMEM((1, BT), jnp.float32),
            pltpu.VMEM((1, BT), jnp.float32),
        ],
    )(x2, e2, x_flat, emb, jnp.asarray(_REVCOL))


def _gather_kernel(table_hbm, idx_hbm, out_hbm, idx_v, rows_v, sem):
    wid = lax.axis_index("s") * _NC + lax.axis_index("c")
    base = wid * _BPW
    pltpu.sync_copy(idx_hbm.at[wid], idx_v)
    copies = []
    for j in range(_NCHUNK):
        copies.append(pltpu.async_copy(
            table_hbm.at[idx_v.at[j]],
            rows_v.at[pl.ds(j * _IDX_CHUNK, _IDX_CHUNK)],
            sem))
    for c in copies:
        c.wait()
    pltpu.sync_copy(rows_v, out_hbm.at[pl.ds(base, _BPW)])


def _gather_call(emb, idx):
    mesh = plsc.VectorSubcoreMesh(core_axis_name="c", subcore_axis_name="s")
    fn = functools.partial(
        pl.kernel,
        mesh=mesh,
        out_type=jax.ShapeDtypeStruct((N_TOK, D_EMBED), jnp.float32),
        scratch_types=[
            pltpu.VMEM((_NCHUNK, _IDX_CHUNK), jnp.int32),
            pltpu.VMEM((_BPW, D_EMBED), jnp.float32),
            pltpu.SemaphoreType.DMA,
        ],
    )(_gather_kernel)
    return fn(emb, idx)


def _transpose_body(q_ref, o_ref):
    o_ref[...] = jnp.transpose(q_ref[...], (0, 2, 1))


def _transpose_call(q3):
    return pl.pallas_call(
        _transpose_body,
        grid=(BATCH,),
        in_specs=[pl.BlockSpec((1, SEQ, D_EMBED), lambda b: (b, 0, 0))],
        out_specs=pl.BlockSpec((1, D_EMBED, SEQ), lambda b: (b, 0, 0)),
        out_shape=jax.ShapeDtypeStruct((BATCH, D_EMBED, SEQ), jnp.float32),
    )(q3)


def kernel(x, embedding_weight):
    xt = jnp.transpose(x, (0, 2, 1))
    x_flat = xt.reshape(-1, D_EMBED)
    x2 = jnp.sum(x_flat ** 2, axis=1, keepdims=True).reshape(1, N_TOK)
    emb_rev = jnp.transpose(
        embedding_weight.reshape((K_EMBED // BK,) + (2,) * 10 + (D_EMBED,)),
        (0, 10, 9, 8, 7, 6, 5, 4, 3, 2, 1, 11)).reshape(K_EMBED, D_EMBED)
    e2 = jnp.sum(emb_rev ** 2, axis=1).reshape(K_EMBED, 1)

    idx2, lsum = _argmin_call(x2, e2, x_flat, emb_rev)

    idx_sc = idx2.reshape(_NW, _NCHUNK, _IDX_CHUNK)
    q_flat = _gather_call(embedding_weight, idx_sc)

    quant = _transpose_call(q_flat.reshape(BATCH, SEQ, D_EMBED))

    loss = (1.0 + COMMIT_W) * lsum[0, 0] / jnp.float32(N_TOK * D_EMBED)
    return (quant, loss)


# speedup vs baseline: 1.1888x; 1.1888x over previous
import functools

import jax
import jax.numpy as jnp
import numpy as np
from jax import lax
from jax.experimental import pallas as pl
from jax.experimental.pallas import tpu as pltpu
from jax.experimental.pallas import tpu_sc as plsc

K_EMBED = 8192
D_EMBED = 256
COMMIT_W = 0.25

BATCH = 8
SEQ = 1024
N_TOK = BATCH * SEQ

BT = 1024
BK = 1024
TB = N_TOK // BT
KB = K_EMBED // BK

_REV = np.zeros(BK, dtype=np.int32)
for _r in range(BK):
    _b = 0
    for _k in range(10):
        _b = (_b << 1) | ((_r >> _k) & 1)
    _REV[_r] = _b
_PERM = (np.arange(K_EMBED, dtype=np.int32) // BK) * BK + np.tile(_REV, K_EMBED // BK)
_REVCOL = _REV.astype(np.float32).reshape(BK, 1)

_NC = 2
_NS = 16
_NW = _NC * _NS
_BPW = N_TOK // _NW
_IDX_CHUNK = 128
_NCHUNK = _BPW // _IDX_CHUNK


def _argmin_body(x2_ref, e2_ref, x_ref, e_ref, rev_ref, idx_ref, lsum_ref,
                 best_ref, bidx_ref):
    tb = pl.program_id(0)
    kb = pl.program_id(1)

    mm = lax.dot_general(
        e_ref[...], x_ref[...],
        dimension_numbers=(((1,), (1,)), ((), ())),
        preferred_element_type=jnp.float32)
    s = x2_ref[...] + e2_ref[...]
    dist = s - 2.0 * mm

    v = dist
    i = jnp.broadcast_to(rev_ref[...], (BK, BT))
    h = BK
    while h > 1:
        h //= 2
        lo_v, hi_v = v[:h], v[h:]
        take = hi_v < lo_v
        v = jnp.where(take, hi_v, lo_v)
        i = jnp.where(take, i[h:], i[:h])
    minv = v
    midx = i + (kb * BK).astype(jnp.float32)

    @pl.when(kb == 0)
    def _():
        best_ref[...] = minv
        bidx_ref[...] = midx

    @pl.when(kb > 0)
    def _():
        upd = minv < best_ref[...]
        best_ref[...] = jnp.where(upd, minv, best_ref[...])
        bidx_ref[...] = jnp.where(upd, midx, bidx_ref[...])

    @pl.when((tb == 0) & (kb == 0))
    def _():
        lsum_ref[...] = jnp.zeros((1, 1), jnp.float32)

    @pl.when(kb == KB - 1)
    def _():
        idx_ref[...] = bidx_ref[...].astype(jnp.int32).reshape(1, 1, BT)
        lsum_ref[...] += jnp.sum(best_ref[...]).reshape(1, 1)


def _argmin_call(x2, e2, x_flat, emb):
    return pl.pallas_call(
        _argmin_body,
        grid=(TB, KB),
        in_specs=[
            pl.BlockSpec((1, BT), lambda tb, kb: (0, tb)),
            pl.BlockSpec((BK, 1), lambda tb, kb: (kb, 0)),
            pl.BlockSpec((BT, D_EMBED), lambda tb, kb: (tb, 0)),
            pl.BlockSpec((BK, D_EMBED), lambda tb, kb: (kb, 0)),
            pl.BlockSpec((BK, 1), lambda tb, kb: (0, 0)),
        ],
        out_specs=[
            pl.BlockSpec((1, 1, BT), lambda tb, kb: (tb, 0, 0)),
            pl.BlockSpec((1, 1), lambda tb, kb: (0, 0)),
        ],
        out_shape=[
            jax.ShapeDtypeStruct((TB, 1, BT), jnp.int32),
            jax.ShapeDtypeStruct((1, 1), jnp.float32),
        ],
        scratch_shapes=[
            pltpu.VMEM((1, BT), jnp.float32),
            pltpu.VMEM((1, BT), jnp.float32),
        ],
    )(x2, e2, x_flat, emb, jnp.asarray(_REVCOL))


def _gather_kernel(table_hbm, idx_hbm, out_hbm, idx_v, rows_v, sem):
    wid = lax.axis_index("s") * _NC + lax.axis_index("c")
    base = wid * _BPW
    pltpu.sync_copy(idx_hbm.at[wid], idx_v)
    copies = []
    for j in range(_NCHUNK):
        copies.append(pltpu.async_copy(
            table_hbm.at[idx_v.at[j]],
            rows_v.at[pl.ds(j * _IDX_CHUNK, _IDX_CHUNK)],
            sem))
    for c in copies:
        c.wait()
    pltpu.sync_copy(rows_v, out_hbm.at[pl.ds(base, _BPW)])


def _gather_call(emb, idx):
    mesh = plsc.VectorSubcoreMesh(core_axis_name="c", subcore_axis_name="s")
    fn = functools.partial(
        pl.kernel,
        mesh=mesh,
        out_type=jax.ShapeDtypeStruct((N_TOK, D_EMBED), jnp.float32),
        scratch_types=[
            pltpu.VMEM((_NCHUNK, _IDX_CHUNK), jnp.int32),
            pltpu.VMEM((_BPW, D_EMBED), jnp.float32),
            pltpu.SemaphoreType.DMA,
        ],
    )(_gather_kernel)
    return fn(emb, idx)


def _transpose_body(q_ref, o_ref):
    o_ref[...] = jnp.transpose(q_ref[...], (0, 2, 1))


def _transpose_call(q3):
    return pl.pallas_call(
        _transpose_body,
        grid=(BATCH,),
        in_specs=[pl.BlockSpec((1, SEQ, D_EMBED), lambda b: (b, 0, 0))],
        out_specs=pl.BlockSpec((1, D_EMBED, SEQ), lambda b: (b, 0, 0)),
        out_shape=jax.ShapeDtypeStruct((BATCH, D_EMBED, SEQ), jnp.float32),
    )(q3)


def kernel(x, embedding_weight):
    xt = jnp.transpose(x, (0, 2, 1))
    x_flat = xt.reshape(-1, D_EMBED)
    x2 = jnp.sum(x_flat ** 2, axis=1, keepdims=True).reshape(1, N_TOK)
    emb_rev = jnp.take(embedding_weight, jnp.asarray(_PERM), axis=0)
    e2 = jnp.sum(emb_rev ** 2, axis=1).reshape(K_EMBED, 1)

    idx2, lsum = _argmin_call(x2, e2, x_flat, emb_rev)

    idx_sc = idx2.reshape(_NW, _NCHUNK, _IDX_CHUNK)
    q_flat = _gather_call(embedding_weight, idx_sc)

    quant = _transpose_call(q_flat.reshape(BATCH, SEQ, D_EMBED))

    loss = (1.0 + COMMIT_W) * lsum[0, 0] / jnp.float32(N_TOK * D_EMBED)
    return (quant, loss)


# speedup vs baseline: 1.2426x; 1.0452x over previous
import functools

import jax
import jax.numpy as jnp
import numpy as np
from jax import lax
from jax.experimental import pallas as pl
from jax.experimental.pallas import tpu as pltpu
from jax.experimental.pallas import tpu_sc as plsc

K_EMBED = 8192
D_EMBED = 256
COMMIT_W = 0.25

BATCH = 8
SEQ = 1024
N_TOK = BATCH * SEQ

BT = 2048
BK = 1024
TB = N_TOK // BT
KB = K_EMBED // BK

_REV = np.zeros(BK, dtype=np.int32)
for _r in range(BK):
    _b = 0
    for _k in range(10):
        _b = (_b << 1) | ((_r >> _k) & 1)
    _REV[_r] = _b
_PERM = (np.arange(K_EMBED, dtype=np.int32) // BK) * BK + np.tile(_REV, K_EMBED // BK)
_REVCOL = _REV.astype(np.float32).reshape(BK, 1)

_NC = 2
_NS = 16
_NW = _NC * _NS
_BPW = N_TOK // _NW
_IDX_CHUNK = 128
_NCHUNK = _BPW // _IDX_CHUNK


def _argmin_body(x2_ref, e2_ref, x_ref, e_ref, rev_ref, idx_ref, lsum_ref,
                 best_ref, bidx_ref):
    tb = pl.program_id(0)
    kb = pl.program_id(1)

    mm = lax.dot_general(
        e_ref[...], x_ref[...],
        dimension_numbers=(((1,), (1,)), ((), ())),
        preferred_element_type=jnp.float32)
    s = x2_ref[...] + e2_ref[...]
    dist = s - 2.0 * mm

    v = dist
    i = jnp.broadcast_to(rev_ref[...], (BK, BT))
    h = BK
    while h > 1:
        h //= 2
        lo_v, hi_v = v[:h], v[h:]
        take = hi_v < lo_v
        v = jnp.where(take, hi_v, lo_v)
        i = jnp.where(take, i[h:], i[:h])
    minv = v
    midx = i + (kb * BK).astype(jnp.float32)

    @pl.when(kb == 0)
    def _():
        best_ref[...] = minv
        bidx_ref[...] = midx

    @pl.when(kb > 0)
    def _():
        upd = minv < best_ref[...]
        best_ref[...] = jnp.where(upd, minv, best_ref[...])
        bidx_ref[...] = jnp.where(upd, midx, bidx_ref[...])

    @pl.when((tb == 0) & (kb == 0))
    def _():
        lsum_ref[...] = jnp.zeros((1, 1), jnp.float32)

    @pl.when(kb == KB - 1)
    def _():
        idx_ref[...] = bidx_ref[...].astype(jnp.int32).reshape(1, 1, BT)
        lsum_ref[...] += jnp.sum(best_ref[...]).reshape(1, 1)


def _argmin_call(x2, e2, x_flat, emb):
    return pl.pallas_call(
        _argmin_body,
        grid=(TB, KB),
        in_specs=[
            pl.BlockSpec((1, BT), lambda tb, kb: (0, tb)),
            pl.BlockSpec((BK, 1), lambda tb, kb: (kb, 0)),
            pl.BlockSpec((BT, D_EMBED), lambda tb, kb: (tb, 0)),
            pl.BlockSpec((BK, D_EMBED), lambda tb, kb: (kb, 0)),
            pl.BlockSpec((BK, 1), lambda tb, kb: (0, 0)),
        ],
        out_specs=[
            pl.BlockSpec((1, 1, BT), lambda tb, kb: (tb, 0, 0)),
            pl.BlockSpec((1, 1), lambda tb, kb: (0, 0)),
        ],
        out_shape=[
            jax.ShapeDtypeStruct((TB, 1, BT), jnp.int32),
            jax.ShapeDtypeStruct((1, 1), jnp.float32),
        ],
        scratch_shapes=[
            pltpu.VMEM((1, BT), jnp.float32),
            pltpu.VMEM((1, BT), jnp.float32),
        ],
    )(x2, e2, x_flat, emb, jnp.asarray(_REVCOL))


def _gather_kernel(table_hbm, idx_hbm, out_hbm, idx_v, rows_v, sem):
    wid = lax.axis_index("s") * _NC + lax.axis_index("c")
    base = wid * _BPW
    pltpu.sync_copy(idx_hbm.at[wid], idx_v)
    copies = []
    for j in range(_NCHUNK):
        copies.append(pltpu.async_copy(
            table_hbm.at[idx_v.at[j]],
            rows_v.at[pl.ds(j * _IDX_CHUNK, _IDX_CHUNK)],
            sem))
    for c in copies:
        c.wait()
    pltpu.sync_copy(rows_v, out_hbm.at[pl.ds(base, _BPW)])


def _gather_call(emb, idx):
    mesh = plsc.VectorSubcoreMesh(core_axis_name="c", subcore_axis_name="s")
    fn = functools.partial(
        pl.kernel,
        mesh=mesh,
        out_type=jax.ShapeDtypeStruct((N_TOK, D_EMBED), jnp.float32),
        scratch_types=[
            pltpu.VMEM((_NCHUNK, _IDX_CHUNK), jnp.int32),
            pltpu.VMEM((_BPW, D_EMBED), jnp.float32),
            pltpu.SemaphoreType.DMA,
        ],
    )(_gather_kernel)
    return fn(emb, idx)


def _transpose_body(q_ref, o_ref):
    o_ref[...] = jnp.transpose(q_ref[...], (0, 2, 1))


def _transpose_call(q3):
    return pl.pallas_call(
        _transpose_body,
        grid=(BATCH,),
        in_specs=[pl.BlockSpec((1, SEQ, D_EMBED), lambda b: (b, 0, 0))],
        out_specs=pl.BlockSpec((1, D_EMBED, SEQ), lambda b: (b, 0, 0)),
        out_shape=jax.ShapeDtypeStruct((BATCH, D_EMBED, SEQ), jnp.float32),
    )(q3)


def kernel(x, embedding_weight):
    xt = jnp.transpose(x, (0, 2, 1))
    x_flat = xt.reshape(-1, D_EMBED)
    x2 = jnp.sum(x_flat ** 2, axis=1, keepdims=True).reshape(1, N_TOK)
    emb_rev = jnp.take(embedding_weight, jnp.asarray(_PERM), axis=0)
    e2 = jnp.sum(emb_rev ** 2, axis=1).reshape(K_EMBED, 1)

    idx2, lsum = _argmin_call(x2, e2, x_flat, emb_rev)

    idx_sc = idx2.reshape(_NW, _NCHUNK, _IDX_CHUNK)
    q_flat = _gather_call(embedding_weight, idx_sc)

    quant = _transpose_call(q_flat.reshape(BATCH, SEQ, D_EMBED))

    loss = (1.0 + COMMIT_W) * lsum[0, 0] / jnp.float32(N_TOK * D_EMBED)
    return (quant, loss)


# speedup vs baseline: 1.2513x; 1.0070x over previous
import functools

import jax
import jax.numpy as jnp
import numpy as np
from jax import lax
from jax.experimental import pallas as pl
from jax.experimental.pallas import tpu as pltpu
from jax.experimental.pallas import tpu_sc as plsc

K_EMBED = 8192
D_EMBED = 256
COMMIT_W = 0.25

BATCH = 8
SEQ = 1024
N_TOK = BATCH * SEQ

BT = 4096
BK = 1024
TB = N_TOK // BT
KB = K_EMBED // BK

_REV = np.zeros(BK, dtype=np.int32)
for _r in range(BK):
    _b = 0
    for _k in range(10):
        _b = (_b << 1) | ((_r >> _k) & 1)
    _REV[_r] = _b
_PERM = (np.arange(K_EMBED, dtype=np.int32) // BK) * BK + np.tile(_REV, K_EMBED // BK)
_REVCOL = _REV.astype(np.float32).reshape(BK, 1)

_NC = 2
_NS = 16
_NW = _NC * _NS
_BPW = N_TOK // _NW
_IDX_CHUNK = 128
_NCHUNK = _BPW // _IDX_CHUNK


def _argmin_body(x2_ref, e2_ref, x_ref, e_ref, rev_ref, idx_ref, lsum_ref,
                 best_ref, bidx_ref):
    tb = pl.program_id(0)
    kb = pl.program_id(1)

    mm = lax.dot_general(
        e_ref[...], x_ref[...],
        dimension_numbers=(((1,), (1,)), ((), ())),
        preferred_element_type=jnp.float32)
    s = x2_ref[...] + e2_ref[...]
    dist = s - 2.0 * mm

    v = dist
    i = jnp.broadcast_to(rev_ref[...], (BK, BT))
    h = BK
    while h > 1:
        h //= 2
        lo_v, hi_v = v[:h], v[h:]
        take = hi_v < lo_v
        v = jnp.where(take, hi_v, lo_v)
        i = jnp.where(take, i[h:], i[:h])
    minv = v
    midx = i + (kb * BK).astype(jnp.float32)

    @pl.when(kb == 0)
    def _():
        best_ref[...] = minv
        bidx_ref[...] = midx

    @pl.when(kb > 0)
    def _():
        upd = minv < best_ref[...]
        best_ref[...] = jnp.where(upd, minv, best_ref[...])
        bidx_ref[...] = jnp.where(upd, midx, bidx_ref[...])

    @pl.when((tb == 0) & (kb == 0))
    def _():
        lsum_ref[...] = jnp.zeros((1, 1), jnp.float32)

    @pl.when(kb == KB - 1)
    def _():
        idx_ref[...] = bidx_ref[...].astype(jnp.int32).reshape(1, 1, BT)
        lsum_ref[...] += jnp.sum(best_ref[...]).reshape(1, 1)


def _argmin_call(x2, e2, x_flat, emb):
    return pl.pallas_call(
        _argmin_body,
        grid=(TB, KB),
        in_specs=[
            pl.BlockSpec((1, BT), lambda tb, kb: (0, tb)),
            pl.BlockSpec((BK, 1), lambda tb, kb: (kb, 0)),
            pl.BlockSpec((BT, D_EMBED), lambda tb, kb: (tb, 0)),
            pl.BlockSpec((BK, D_EMBED), lambda tb, kb: (kb, 0)),
            pl.BlockSpec((BK, 1), lambda tb, kb: (0, 0)),
        ],
        out_specs=[
            pl.BlockSpec((1, 1, BT), lambda tb, kb: (tb, 0, 0)),
            pl.BlockSpec((1, 1), lambda tb, kb: (0, 0)),
        ],
        out_shape=[
            jax.ShapeDtypeStruct((TB, 1, BT), jnp.int32),
            jax.ShapeDtypeStruct((1, 1), jnp.float32),
        ],
        scratch_shapes=[
            pltpu.VMEM((1, BT), jnp.float32),
            pltpu.VMEM((1, BT), jnp.float32),
        ],
    )(x2, e2, x_flat, emb, jnp.asarray(_REVCOL))


def _gather_kernel(table_hbm, idx_hbm, out_hbm, idx_v, rows_v, sem):
    wid = lax.axis_index("s") * _NC + lax.axis_index("c")
    base = wid * _BPW
    pltpu.sync_copy(idx_hbm.at[wid], idx_v)
    copies = []
    for j in range(_NCHUNK):
        copies.append(pltpu.async_copy(
            table_hbm.at[idx_v.at[j]],
            rows_v.at[pl.ds(j * _IDX_CHUNK, _IDX_CHUNK)],
            sem))
    for c in copies:
        c.wait()
    pltpu.sync_copy(rows_v, out_hbm.at[pl.ds(base, _BPW)])


def _gather_call(emb, idx):
    mesh = plsc.VectorSubcoreMesh(core_axis_name="c", subcore_axis_name="s")
    fn = functools.partial(
        pl.kernel,
        mesh=mesh,
        out_type=jax.ShapeDtypeStruct((N_TOK, D_EMBED), jnp.float32),
        scratch_types=[
            pltpu.VMEM((_NCHUNK, _IDX_CHUNK), jnp.int32),
            pltpu.VMEM((_BPW, D_EMBED), jnp.float32),
            pltpu.SemaphoreType.DMA,
        ],
    )(_gather_kernel)
    return fn(emb, idx)


def _transpose_body(q_ref, o_ref):
    o_ref[...] = jnp.transpose(q_ref[...], (0, 2, 1))


def _transpose_call(q3):
    return pl.pallas_call(
        _transpose_body,
        grid=(BATCH,),
        in_specs=[pl.BlockSpec((1, SEQ, D_EMBED), lambda b: (b, 0, 0))],
        out_specs=pl.BlockSpec((1, D_EMBED, SEQ), lambda b: (b, 0, 0)),
        out_shape=jax.ShapeDtypeStruct((BATCH, D_EMBED, SEQ), jnp.float32),
    )(q3)


def kernel(x, embedding_weight):
    xt = jnp.transpose(x, (0, 2, 1))
    x_flat = xt.reshape(-1, D_EMBED)
    x2 = jnp.sum(x_flat ** 2, axis=1, keepdims=True).reshape(1, N_TOK)
    emb_rev = jnp.take(embedding_weight, jnp.asarray(_PERM), axis=0)
    e2 = jnp.sum(emb_rev ** 2, axis=1).reshape(K_EMBED, 1)

    idx2, lsum = _argmin_call(x2, e2, x_flat, emb_rev)

    idx_sc = idx2.reshape(_NW, _NCHUNK, _IDX_CHUNK)
    q_flat = _gather_call(embedding_weight, idx_sc)

    quant = _transpose_call(q_flat.reshape(BATCH, SEQ, D_EMBED))

    loss = (1.0 + COMMIT_W) * lsum[0, 0] / jnp.float32(N_TOK * D_EMBED)
    return (quant, loss)


# speedup vs baseline: 1.2550x; 1.0030x over previous
import functools

import jax
import jax.numpy as jnp
import numpy as np
from jax import lax
from jax.experimental import pallas as pl
from jax.experimental.pallas import tpu as pltpu
from jax.experimental.pallas import tpu_sc as plsc

K_EMBED = 8192
D_EMBED = 256
COMMIT_W = 0.25

BATCH = 8
SEQ = 1024
N_TOK = BATCH * SEQ

BT = 8192
BK = 1024
TB = N_TOK // BT
KB = K_EMBED // BK

_REV = np.zeros(BK, dtype=np.int32)
for _r in range(BK):
    _b = 0
    for _k in range(10):
        _b = (_b << 1) | ((_r >> _k) & 1)
    _REV[_r] = _b
_PERM = (np.arange(K_EMBED, dtype=np.int32) // BK) * BK + np.tile(_REV, K_EMBED // BK)
_REVCOL = _REV.astype(np.float32).reshape(BK, 1)

_NC = 2
_NS = 16
_NW = _NC * _NS
_BPW = N_TOK // _NW
_IDX_CHUNK = 128
_NCHUNK = _BPW // _IDX_CHUNK


def _argmin_body(x2_ref, e2_ref, x_ref, e_ref, rev_ref, idx_ref, lsum_ref,
                 best_ref, bidx_ref):
    tb = pl.program_id(0)
    kb = pl.program_id(1)

    mm = lax.dot_general(
        e_ref[...], x_ref[...],
        dimension_numbers=(((1,), (1,)), ((), ())),
        preferred_element_type=jnp.float32)
    s = x2_ref[...] + e2_ref[...]
    dist = s - 2.0 * mm

    v = dist
    i = jnp.broadcast_to(rev_ref[...], (BK, BT))
    h = BK
    while h > 1:
        h //= 2
        lo_v, hi_v = v[:h], v[h:]
        take = hi_v < lo_v
        v = jnp.where(take, hi_v, lo_v)
        i = jnp.where(take, i[h:], i[:h])
    minv = v
    midx = i + (kb * BK).astype(jnp.float32)

    @pl.when(kb == 0)
    def _():
        best_ref[...] = minv
        bidx_ref[...] = midx

    @pl.when(kb > 0)
    def _():
        upd = minv < best_ref[...]
        best_ref[...] = jnp.where(upd, minv, best_ref[...])
        bidx_ref[...] = jnp.where(upd, midx, bidx_ref[...])

    @pl.when((tb == 0) & (kb == 0))
    def _():
        lsum_ref[...] = jnp.zeros((1, 1), jnp.float32)

    @pl.when(kb == KB - 1)
    def _():
        idx_ref[...] = bidx_ref[...].astype(jnp.int32).reshape(1, 1, BT)
        lsum_ref[...] += jnp.sum(best_ref[...]).reshape(1, 1)


def _argmin_call(x2, e2, x_flat, emb):
    return pl.pallas_call(
        _argmin_body,
        grid=(TB, KB),
        in_specs=[
            pl.BlockSpec((1, BT), lambda tb, kb: (0, tb)),
            pl.BlockSpec((BK, 1), lambda tb, kb: (kb, 0)),
            pl.BlockSpec((BT, D_EMBED), lambda tb, kb: (tb, 0)),
            pl.BlockSpec((BK, D_EMBED), lambda tb, kb: (kb, 0)),
            pl.BlockSpec((BK, 1), lambda tb, kb: (0, 0)),
        ],
        out_specs=[
            pl.BlockSpec((1, 1, BT), lambda tb, kb: (tb, 0, 0)),
            pl.BlockSpec((1, 1), lambda tb, kb: (0, 0)),
        ],
        out_shape=[
            jax.ShapeDtypeStruct((TB, 1, BT), jnp.int32),
            jax.ShapeDtypeStruct((1, 1), jnp.float32),
        ],
        scratch_shapes=[
            pltpu.VMEM((1, BT), jnp.float32),
            pltpu.VMEM((1, BT), jnp.float32),
        ],
    )(x2, e2, x_flat, emb, jnp.asarray(_REVCOL))


def _gather_kernel(table_hbm, idx_hbm, out_hbm, idx_v, rows_v, sem):
    wid = lax.axis_index("s") * _NC + lax.axis_index("c")
    base = wid * _BPW
    pltpu.sync_copy(idx_hbm.at[wid], idx_v)
    copies = []
    for j in range(_NCHUNK):
        copies.append(pltpu.async_copy(
            table_hbm.at[idx_v.at[j]],
            rows_v.at[pl.ds(j * _IDX_CHUNK, _IDX_CHUNK)],
            sem))
    for c in copies:
        c.wait()
    pltpu.sync_copy(rows_v, out_hbm.at[pl.ds(base, _BPW)])


def _gather_call(emb, idx):
    mesh = plsc.VectorSubcoreMesh(core_axis_name="c", subcore_axis_name="s")
    fn = functools.partial(
        pl.kernel,
        mesh=mesh,
        out_type=jax.ShapeDtypeStruct((N_TOK, D_EMBED), jnp.float32),
        scratch_types=[
            pltpu.VMEM((_NCHUNK, _IDX_CHUNK), jnp.int32),
            pltpu.VMEM((_BPW, D_EMBED), jnp.float32),
            pltpu.SemaphoreType.DMA,
        ],
    )(_gather_kernel)
    return fn(emb, idx)


def _transpose_body(q_ref, o_ref):
    o_ref[...] = jnp.transpose(q_ref[...], (0, 2, 1))


def _transpose_call(q3):
    return pl.pallas_call(
        _transpose_body,
        grid=(BATCH,),
        in_specs=[pl.BlockSpec((1, SEQ, D_EMBED), lambda b: (b, 0, 0))],
        out_specs=pl.BlockSpec((1, D_EMBED, SEQ), lambda b: (b, 0, 0)),
        out_shape=jax.ShapeDtypeStruct((BATCH, D_EMBED, SEQ), jnp.float32),
    )(q3)


def kernel(x, embedding_weight):
    xt = jnp.transpose(x, (0, 2, 1))
    x_flat = xt.reshape(-1, D_EMBED)
    x2 = jnp.sum(x_flat ** 2, axis=1, keepdims=True).reshape(1, N_TOK)
    emb_rev = jnp.take(embedding_weight, jnp.asarray(_PERM), axis=0)
    e2 = jnp.sum(emb_rev ** 2, axis=1).reshape(K_EMBED, 1)

    idx2, lsum = _argmin_call(x2, e2, x_flat, emb_rev)

    idx_sc = idx2.reshape(_NW, _NCHUNK, _IDX_CHUNK)
    q_flat = _gather_call(embedding_weight, idx_sc)

    quant = _transpose_call(q_flat.reshape(BATCH, SEQ, D_EMBED))

    loss = (1.0 + COMMIT_W) * lsum[0, 0] / jnp.float32(N_TOK * D_EMBED)
    return (quant, loss)


# speedup vs baseline: 1.2996x; 1.0355x over previous
import functools

import jax
import jax.numpy as jnp
import numpy as np
from jax import lax
from jax.experimental import pallas as pl
from jax.experimental.pallas import tpu as pltpu
from jax.experimental.pallas import tpu_sc as plsc

K_EMBED = 8192
D_EMBED = 256
COMMIT_W = 0.25

BATCH = 8
SEQ = 1024
N_TOK = BATCH * SEQ

BT = 8192
BK = 2048
TB = N_TOK // BT
KB = K_EMBED // BK

_NBITS = BK.bit_length() - 1
_REV = np.zeros(BK, dtype=np.int32)
for _r in range(BK):
    _b = 0
    for _k in range(_NBITS):
        _b = (_b << 1) | ((_r >> _k) & 1)
    _REV[_r] = _b
_PERM = (np.arange(K_EMBED, dtype=np.int32) // BK) * BK + np.tile(_REV, K_EMBED // BK)
_REVCOL = _REV.astype(np.float32).reshape(BK, 1)

_NC = 2
_NS = 16
_NW = _NC * _NS
_BPW = N_TOK // _NW
_IDX_CHUNK = 128
_NCHUNK = _BPW // _IDX_CHUNK


def _argmin_body(x2_ref, e2_ref, x_ref, e_ref, rev_ref, idx_ref, lsum_ref,
                 best_ref, bidx_ref):
    tb = pl.program_id(0)
    kb = pl.program_id(1)

    mm = lax.dot_general(
        e_ref[...], x_ref[...],
        dimension_numbers=(((1,), (1,)), ((), ())),
        preferred_element_type=jnp.float32)
    s = x2_ref[...] + e2_ref[...]
    dist = s - 2.0 * mm

    v = dist
    i = jnp.broadcast_to(rev_ref[...], (BK, BT))
    h = BK
    while h > 1:
        h //= 2
        lo_v, hi_v = v[:h], v[h:]
        take = hi_v < lo_v
        v = jnp.where(take, hi_v, lo_v)
        i = jnp.where(take, i[h:], i[:h])
    minv = v
    midx = i + (kb * BK).astype(jnp.float32)

    @pl.when(kb == 0)
    def _():
        best_ref[...] = minv
        bidx_ref[...] = midx

    @pl.when(kb > 0)
    def _():
        upd = minv < best_ref[...]
        best_ref[...] = jnp.where(upd, minv, best_ref[...])
        bidx_ref[...] = jnp.where(upd, midx, bidx_ref[...])

    @pl.when((tb == 0) & (kb == 0))
    def _():
        lsum_ref[...] = jnp.zeros((1, 1), jnp.float32)

    @pl.when(kb == KB - 1)
    def _():
        idx_ref[...] = bidx_ref[...].astype(jnp.int32).reshape(1, 1, BT)
        lsum_ref[...] += jnp.sum(best_ref[...]).reshape(1, 1)


def _argmin_call(x2, e2, x_flat, emb):
    return pl.pallas_call(
        _argmin_body,
        grid=(TB, KB),
        in_specs=[
            pl.BlockSpec((1, BT), lambda tb, kb: (0, tb)),
            pl.BlockSpec((BK, 1), lambda tb, kb: (kb, 0)),
            pl.BlockSpec((BT, D_EMBED), lambda tb, kb: (tb, 0)),
            pl.BlockSpec((BK, D_EMBED), lambda tb, kb: (kb, 0)),
            pl.BlockSpec((BK, 1), lambda tb, kb: (0, 0)),
        ],
        out_specs=[
            pl.BlockSpec((1, 1, BT), lambda tb, kb: (tb, 0, 0)),
            pl.BlockSpec((1, 1), lambda tb, kb: (0, 0)),
        ],
        out_shape=[
            jax.ShapeDtypeStruct((TB, 1, BT), jnp.int32),
            jax.ShapeDtypeStruct((1, 1), jnp.float32),
        ],
        scratch_shapes=[
            pltpu.VMEM((1, BT), jnp.float32),
            pltpu.VMEM((1, BT), jnp.float32),
        ],
    )(x2, e2, x_flat, emb, jnp.asarray(_REVCOL))


def _gather_kernel(table_hbm, idx_hbm, out_hbm, idx_v, rows_v, sem):
    wid = lax.axis_index("s") * _NC + lax.axis_index("c")
    base = wid * _BPW
    pltpu.sync_copy(idx_hbm.at[wid], idx_v)
    copies = []
    for j in range(_NCHUNK):
        copies.append(pltpu.async_copy(
            table_hbm.at[idx_v.at[j]],
            rows_v.at[pl.ds(j * _IDX_CHUNK, _IDX_CHUNK)],
            sem))
    for c in copies:
        c.wait()
    pltpu.sync_copy(rows_v, out_hbm.at[pl.ds(base, _BPW)])


def _gather_call(emb, idx):
    mesh = plsc.VectorSubcoreMesh(core_axis_name="c", subcore_axis_name="s")
    fn = functools.partial(
        pl.kernel,
        mesh=mesh,
        out_type=jax.ShapeDtypeStruct((N_TOK, D_EMBED), jnp.float32),
        scratch_types=[
            pltpu.VMEM((_NCHUNK, _IDX_CHUNK), jnp.int32),
            pltpu.VMEM((_BPW, D_EMBED), jnp.float32),
            pltpu.SemaphoreType.DMA,
        ],
    )(_gather_kernel)
    return fn(emb, idx)


def _transpose_body(q_ref, o_ref):
    o_ref[...] = jnp.transpose(q_ref[...], (0, 2, 1))


def _transpose_call(q3):
    return pl.pallas_call(
        _transpose_body,
        grid=(BATCH,),
        in_specs=[pl.BlockSpec((1, SEQ, D_EMBED), lambda b: (b, 0, 0))],
        out_specs=pl.BlockSpec((1, D_EMBED, SEQ), lambda b: (b, 0, 0)),
        out_shape=jax.ShapeDtypeStruct((BATCH, D_EMBED, SEQ), jnp.float32),
    )(q3)


def kernel(x, embedding_weight):
    xt = jnp.transpose(x, (0, 2, 1))
    x_flat = xt.reshape(-1, D_EMBED)
    x2 = jnp.sum(x_flat ** 2, axis=1, keepdims=True).reshape(1, N_TOK)
    emb_rev = jnp.take(embedding_weight, jnp.asarray(_PERM), axis=0)
    e2 = jnp.sum(emb_rev ** 2, axis=1).reshape(K_EMBED, 1)

    idx2, lsum = _argmin_call(x2, e2, x_flat, emb_rev)

    idx_sc = idx2.reshape(_NW, _NCHUNK, _IDX_CHUNK)
    q_flat = _gather_call(embedding_weight, idx_sc)

    quant = _transpose_call(q_flat.reshape(BATCH, SEQ, D_EMBED))

    loss = (1.0 + COMMIT_W) * lsum[0, 0] / jnp.float32(N_TOK * D_EMBED)
    return (quant, loss)
